# baseline (speedup 1.0000x reference)
import functools

import jax
import jax.numpy as jnp
from jax import lax
from jax.experimental import pallas as pl
from jax.experimental.pallas import tpu as pltpu
from jax.experimental.pallas import tpu_sc as plsc

W_SIZE = 5
STRIDE = 4
PADDING = 2
RIGHT_EXTRA = 1

_NW = 32
_CH = 128
_LANES = 16
_RB = 8


def _transpose_to_rows(x):
    N, C, H, W = x.shape
    br = _RB * W
    nblk = (N * H) // _RB
    hb = H // _RB

    def body(x_ref, o_ref):
        i = pl.program_id(0)

        @pl.when(i < nblk)
        def _():
            blk = x_ref[0].reshape(C, br)
            o_ref[...] = jnp.transpose(blk, (1, 0))

        @pl.when(i == nblk)
        def _():
            o_ref[...] = jnp.zeros((br, C), jnp.float32)

    in_spec = pl.BlockSpec(
        (1, C, _RB, W),
        lambda i: (jnp.where(i < nblk, i // hb, 0), 0,
                   jnp.where(i < nblk, i % hb, 0), 0))
    out_spec = pl.BlockSpec((br, C), lambda i: (i, 0))
    return pl.pallas_call(
        body,
        grid=(nblk + 1,),
        in_specs=[in_spec],
        out_specs=out_spec,
        out_shape=jax.ShapeDtypeStruct(((nblk + 1) * br, C), jnp.float32),
    )(x)


def _build_sc_gather(M, C, H, W, ow, w0, w1, p0, p1, zr):
    k0 = w0 * w0
    k1 = w1 * w1
    hw = H * W
    mpw = -(-M // _NW)
    mpw = -(-mpw // 8) * 8
    mpad = mpw * _NW
    r0 = mpw * k0
    r1 = mpw * k1
    nf0, t0 = divmod(r0, _CH)
    nf1, t1 = divmod(r1, _CH)
    groups = mpw // _LANES

    assert ow & (ow - 1) == 0
    ow_shift = ow.bit_length() - 1
    oh = ow
    assert (oh - 1) * STRIDE + (w0 - 1) - p0 < H
    assert (oh - 1) * STRIDE + (w1 - 1) - p1 < H

    mesh = plsc.VectorSubcoreMesh(core_axis_name="c", subcore_axis_name="s")

    @functools.partial(
        pl.kernel,
        mesh=mesh,
        compiler_params=pltpu.CompilerParams(needs_layout_passes=False),
        out_type=(
            jax.ShapeDtypeStruct((mpad * k0, C), jnp.float32),
            jax.ShapeDtypeStruct((mpad * k1, C), jnp.float32),
        ),
        scratch_types=[
            pltpu.VMEM((mpw,), jnp.int32),
            pltpu.VMEM((mpw,), jnp.int32),
            pltpu.VMEM((mpw,), jnp.int32),
            pltpu.VMEM(((nf0 + 1) * _CH,), jnp.int32),
            pltpu.VMEM(((nf1 + 1) * _CH,), jnp.int32),
            pltpu.VMEM((_CH, C), jnp.float32),
            pltpu.VMEM((_CH, C), jnp.float32),
            pltpu.VMEM((_CH, C), jnp.float32),
            pltpu.VMEM((_CH, C), jnp.float32),
            pltpu.SemaphoreType.DMA,
            pltpu.SemaphoreType.DMA,
            pltpu.SemaphoreType.DMA,
            pltpu.SemaphoreType.DMA,
        ],
    )
    def sc_fn(tab0, tab1, b_hbm, i_hbm, j_hbm, out0, out1,
              b_v, i_v, j_v, idx0_v, idx1_v,
              rows0, rows1, rows2, rows3, sem0, sem1, sem2, sem3):
        rows = (rows0, rows1, rows2, rows3)
        sems = (sem0, sem1, sem2, sem3)
        wid = lax.axis_index("s") * 2 + lax.axis_index("c")
        mbase = wid * mpw
        pltpu.sync_copy(b_hbm.at[pl.ds(mbase, mpw)], b_v)
        pltpu.sync_copy(i_hbm.at[pl.ds(mbase, mpw)], i_v)
        pltpu.sync_copy(j_hbm.at[pl.ds(mbase, mpw)], j_v)

        zeros = jnp.zeros((_LANES,), jnp.int32)
        for s_ in range(0, _CH, _LANES):
            idx0_v[pl.ds(nf0 * _CH + s_, _LANES)] = zeros
            idx1_v[pl.ds(nf1 * _CH + s_, _LANES)] = zeros

        lanes = lax.iota(jnp.int32, _LANES)
        zrv = jnp.full((_LANES,), zr, jnp.int32)

        def fill(g, carry):
            m0 = g * _LANES
            bv = b_v[pl.ds(m0, _LANES)]
            iv = i_v[pl.ds(m0, _LANES)]
            jv = j_v[pl.ds(m0, _LANES)]
            ml = m0 + lanes

            def emit(idx_v, posv, k, w, p, oyp, oxp, basev):
                dy, dx = k // w, k % w
                row = basev + (dy - p) * W + (dx - p)
                need_y = dy < p
                need_x = dx < p
                if need_y and need_x:
                    row = jnp.where(oyp & oxp, row, zrv)
                elif need_y:
                    row = jnp.where(oyp, row, zrv)
                elif need_x:
                    row = jnp.where(oxp, row, zrv)
                plsc.store_scatter(idx_v, [posv], row)

            oy = iv >> ow_shift
            ox = iv & (ow - 1)
            base0 = bv * hw + oy * (STRIDE * W) + ox * STRIDE
            oyp0 = oy > 0
            oxp0 = ox > 0
            for k in range(k0):
                emit(idx0_v, ml * k0 + k, k, w0, p0, oyp0, oxp0, base0)

            oy = jv >> ow_shift
            ox = jv & (ow - 1)
            base1 = bv * hw + oy * (STRIDE * W) + ox * STRIDE
            oyp1 = oy > 0
            oxp1 = ox > 0
            for k in range(k1):
                emit(idx1_v, ml * k1 + k, k, w1, p1, oyp1, oxp1, base1)
            return carry

        lax.fori_loop(0, groups, fill, 0)

        def gather_all(tab, idx_v, out, obase, nfull, tail, pending):
            def wait64(b):
                pltpu.make_async_copy(out.at[pl.ds(obase, _CH)],
                                      rows[b], sems[b]).wait()

            def fire_gather(c, b):
                off = pl.multiple_of(c * _CH, _CH)
                pltpu.async_copy(tab.at[idx_v.at[pl.ds(off, _CH)]],
                                 rows[b], sems[b])

            def fire_write(c, b):
                pltpu.async_copy(rows[b],
                                 out.at[pl.ds(obase + c * _CH, _CH)],
                                 sems[b])

            for b in range(4):
                if pending[b]:
                    wait64(b)
                fire_gather(b, b)

            nq, _rem = divmod(nfull, 4)

            def quad(q, carry):
                for b in range(4):
                    c = q * 4 + b
                    wait64(b)
                    fire_write(c, b)

                    @pl.when(c + 4 < nfull)
                    def _():
                        wait64(b)
                        fire_gather(c + 4, b)
                return carry

            lax.fori_loop(0, nq, quad, 0)
            for c in range(4 * nq, nfull):
                b = c % 4
                wait64(b)
                fire_write(c, b)
            wait64(0)
            fire_gather(nfull, 0)
            wait64(0)
            pltpu.sync_copy(rows[0].at[pl.ds(0, tail)],
                            out.at[pl.ds(obase + nfull * _CH, tail)])
            return (False, True, True, True)

        pend = gather_all(tab0, idx0_v, out0, wid * r0, nf0, t0,
                          (False,) * 4)
        pend = gather_all(tab1, idx1_v, out1, wid * r1, nf1, t1, pend)
        for b in range(4):
            if pend[b]:
                pltpu.make_async_copy(out1.at[pl.ds(wid * r1, _CH)],
                                      rows[b], sems[b]).wait()

    return mpad, sc_fn


def kernel(x0, x1, b_idxes, i_idxes, j_idxes):
    w0 = W_SIZE
    e = RIGHT_EXTRA
    w1 = w0 + 2 * e
    p0 = PADDING
    p1 = PADDING + e
    N, C, H, W = x0.shape
    ow = (W + 2 * p0 - w0) // STRIDE + 1
    M = b_idxes.shape[0]

    t0 = _transpose_to_rows(x0)
    t1 = _transpose_to_rows(x1)
    zr = N * H * W

    mpad, sc_fn = _build_sc_gather(M, C, H, W, ow, w0, w1, p0, p1, zr)
    pad = mpad - M
    b = jnp.pad(b_idxes.astype(jnp.int32), (0, pad))
    ii = jnp.pad(i_idxes.astype(jnp.int32), (0, pad))
    jj = jnp.pad(j_idxes.astype(jnp.int32), (0, pad))

    out0f, out1f = sc_fn(t0, t1, b, ii, jj)
    out0 = out0f.reshape(mpad, w0 * w0, C)[:M]
    out1 = out1f.reshape(mpad, w1 * w1, C)[:M]
    return out0, out1


# speedup vs baseline: 1.4304x; 1.4304x over previous
import functools

import jax
import jax.numpy as jnp
from jax import lax
from jax.experimental import pallas as pl
from jax.experimental.pallas import tpu as pltpu
from jax.experimental.pallas import tpu_sc as plsc

W_SIZE = 5
STRIDE = 4
PADDING = 2
RIGHT_EXTRA = 1

_NW = 32
_CH = 128
_LANES = 16
_RB = 8


def _transpose_to_rows(x):
    N, C, H, W = x.shape
    br = _RB * W
    nblk = (N * H) // _RB
    hb = H // _RB

    def body(x_ref, o_ref):
        i = pl.program_id(0)

        @pl.when(i < nblk)
        def _():
            blk = x_ref[0].reshape(C, br)
            o_ref[...] = jnp.transpose(blk, (1, 0))

        @pl.when(i == nblk)
        def _():
            o_ref[...] = jnp.zeros((br, C), jnp.float32)

    in_spec = pl.BlockSpec(
        (1, C, _RB, W),
        lambda i: (jnp.where(i < nblk, i // hb, 0), 0,
                   jnp.where(i < nblk, i % hb, 0), 0))
    out_spec = pl.BlockSpec((br, C), lambda i: (i, 0))
    return pl.pallas_call(
        body,
        grid=(nblk + 1,),
        in_specs=[in_spec],
        out_specs=out_spec,
        out_shape=jax.ShapeDtypeStruct(((nblk + 1) * br, C), jnp.float32),
    )(x)


def _build_sc_gather(M, C, H, W, ow, w0, w1, p0, p1, zr):
    k0 = w0 * w0
    k1 = w1 * w1
    hw = H * W
    nfc0, tail0 = divmod(M * k0, _CH)
    nfc1, tail1 = divmod(M * k1, _CH)
    per0 = -(-nfc0 // _NW)
    per1 = -(-nfc1 // _NW)
    mp8 = -(-M // 8) * 8

    assert ow & (ow - 1) == 0
    ow_shift = ow.bit_length() - 1
    oh = ow
    assert (oh - 1) * STRIDE + (w0 - 1) - p0 < H
    assert (oh - 1) * STRIDE + (w1 - 1) - p1 < H

    mesh = plsc.VectorSubcoreMesh(core_axis_name="c", subcore_axis_name="s")

    @functools.partial(
        pl.kernel,
        mesh=mesh,
        compiler_params=pltpu.CompilerParams(needs_layout_passes=False),
        out_type=(
            jax.ShapeDtypeStruct((M * k0, C), jnp.float32),
            jax.ShapeDtypeStruct((M * k1, C), jnp.float32),
        ),
        scratch_types=[
            pltpu.VMEM((mp8,), jnp.int32),
            pltpu.VMEM((mp8,), jnp.int32),
            pltpu.VMEM((mp8,), jnp.int32),
            pltpu.VMEM((_CH,), jnp.int32),
            pltpu.VMEM((_CH,), jnp.int32),
            pltpu.VMEM((_CH,), jnp.int32),
            pltpu.VMEM((_CH,), jnp.int32),
            pltpu.VMEM((_CH, C), jnp.float32),
            pltpu.VMEM((_CH, C), jnp.float32),
            pltpu.VMEM((_CH, C), jnp.float32),
            pltpu.VMEM((_CH, C), jnp.float32),
            pltpu.SemaphoreType.DMA,
            pltpu.SemaphoreType.DMA,
            pltpu.SemaphoreType.DMA,
            pltpu.SemaphoreType.DMA,
        ],
    )
    def sc_fn(tab0, tab1, b_hbm, i_hbm, j_hbm, out0, out1,
              b_v, i_v, j_v, ix0, ix1, ix2, ix3,
              rows0, rows1, rows2, rows3, sem0, sem1, sem2, sem3):
        idxb = (ix0, ix1, ix2, ix3)
        rows = (rows0, rows1, rows2, rows3)
        sems = (sem0, sem1, sem2, sem3)
        wid = lax.axis_index("s") * 2 + lax.axis_index("c")
        lanes = lax.iota(jnp.int32, _LANES)
        zrv = jnp.full((_LANES,), zr, jnp.int32)
        mmax = jnp.full((_LANES,), M - 1, jnp.int32)

        pltpu.sync_copy(b_hbm, b_v)
        pltpu.sync_copy(i_hbm, i_v)
        pltpu.sync_copy(j_hbm, j_v)

        def fill_chunk(buf, c, kx, w, p, pos_v):
            kmax = jnp.full((_LANES,), kx - 1, jnp.int32)

            def grp(g, carry):
                r = c * _CH + g * _LANES + lanes
                m = jnp.minimum(r // kx, mmax)
                kk = jnp.minimum(r - m * kx, kmax)
                bv = plsc.load_gather(b_v, [m])
                pv = plsc.load_gather(pos_v, [m])
                oy = pv >> ow_shift
                ox = pv & (ow - 1)
                dy = kk // w
                dx = kk - dy * w
                y0 = oy * STRIDE + (dy - p)
                x0 = ox * STRIDE + (dx - p)
                valid = (y0 >= 0) & (x0 >= 0)
                row = jnp.where(valid, bv * hw + y0 * W + x0, zrv)
                buf[pl.ds(g * _LANES, _LANES)] = row
                return carry

            lax.fori_loop(0, _CH // _LANES, grp, 0)

        @pl.when(wid == 0)
        def _():
            for (tab, out, pos_v, kx, w, p, nfc, tail) in (
                    (tab0, out0, i_v, k0, w0, p0, nfc0, tail0),
                    (tab1, out1, j_v, k1, w1, p1, nfc1, tail1)):
                fill_chunk(ix0, nfc, kx, w, p, pos_v)
                pltpu.async_copy(tab.at[ix0], rows[0], sems[0]).wait()
                pltpu.sync_copy(rows[0].at[pl.ds(0, tail)],
                                out.at[pl.ds(nfc * _CH, tail)])

        def gather_all(tab, out, pos_v, kx, w, p, nfc, per, pending):
            start = (wid * (nfc - per)) // (_NW - 1)

            def wait64(b):
                pltpu.make_async_copy(out.at[pl.ds(0, _CH)],
                                      rows[b], sems[b]).wait()

            def fire_gather(c, b):
                pltpu.async_copy(tab.at[idxb[b]], rows[b], sems[b])

            def fire_write(c, b):
                off = pl.multiple_of((start + c) * _CH, _CH)
                pltpu.async_copy(rows[b], out.at[pl.ds(off, _CH)], sems[b])

            for b in range(4):
                if pending[b]:
                    wait64(b)
                fill_chunk(idxb[b], start + b, kx, w, p, pos_v)
                fire_gather(b, b)

            nq, _rem = divmod(per, 4)

            def quad(q, carry):
                for b in range(4):
                    c = q * 4 + b
                    wait64(b)
                    fire_write(c, b)

                    @pl.when(c + 4 < per)
                    def _():
                        wait64(b)
                        fill_chunk(idxb[b], start + c + 4, kx, w, p, pos_v)
                        fire_gather(c + 4, b)
                return carry

            lax.fori_loop(0, nq, quad, 0)
            for c in range(4 * nq, per):
                b = c % 4
                wait64(b)
                fire_write(c, b)
            return (True, True, True, True)

        pend = gather_all(tab0, out0, i_v, k0, w0, p0, nfc0, per0,
                          (False,) * 4)
        pend = gather_all(tab1, out1, j_v, k1, w1, p1, nfc1, per1, pend)
        for b in range(4):
            if pend[b]:
                pltpu.make_async_copy(out1.at[pl.ds(0, _CH)],
                                      rows[b], sems[b]).wait()

    return mp8, sc_fn


def kernel(x0, x1, b_idxes, i_idxes, j_idxes):
    w0 = W_SIZE
    e = RIGHT_EXTRA
    w1 = w0 + 2 * e
    p0 = PADDING
    p1 = PADDING + e
    N, C, H, W = x0.shape
    ow = (W + 2 * p0 - w0) // STRIDE + 1
    M = b_idxes.shape[0]

    t0 = _transpose_to_rows(x0)
    t1 = _transpose_to_rows(x1)
    zr = N * H * W

    mp8, sc_fn = _build_sc_gather(M, C, H, W, ow, w0, w1, p0, p1, zr)
    pad = mp8 - M
    b = jnp.pad(b_idxes.astype(jnp.int32), (0, pad))
    ii = jnp.pad(i_idxes.astype(jnp.int32), (0, pad))
    jj = jnp.pad(j_idxes.astype(jnp.int32), (0, pad))

    out0f, out1f = sc_fn(t0, t1, b, ii, jj)
    return (out0f.reshape(M, w0 * w0, C),
            out1f.reshape(M, w1 * w1, C))


# speedup vs baseline: 1.4322x; 1.0013x over previous
import functools

import jax
import jax.numpy as jnp
from jax import lax
from jax.experimental import pallas as pl
from jax.experimental.pallas import tpu as pltpu
from jax.experimental.pallas import tpu_sc as plsc

W_SIZE = 5
STRIDE = 4
PADDING = 2
RIGHT_EXTRA = 1

_NW = 32
_CH = 128
_LANES = 16
_RB = 8


def _transpose_to_rows(x):
    N, C, H, W = x.shape
    br = _RB * W
    nblk = (N * H) // _RB
    hb = H // _RB

    def body(x_ref, o_ref):
        i = pl.program_id(0)

        @pl.when(i < nblk)
        def _():
            blk = x_ref[0].reshape(C, br)
            o_ref[...] = jnp.transpose(blk, (1, 0))

        @pl.when(i == nblk)
        def _():
            o_ref[...] = jnp.zeros((br, C), jnp.float32)

    in_spec = pl.BlockSpec(
        (1, C, _RB, W),
        lambda i: (jnp.where(i < nblk, i // hb, 0), 0,
                   jnp.where(i < nblk, i % hb, 0), 0))
    out_spec = pl.BlockSpec((br, C), lambda i: (i, 0))
    return pl.pallas_call(
        body,
        grid=(nblk + 1,),
        in_specs=[in_spec],
        out_specs=out_spec,
        out_shape=jax.ShapeDtypeStruct(((nblk + 1) * br, C), jnp.float32),
    )(x)


def _build_sc_gather(M, C, H, W, ow, w0, w1, p0, p1, zr):
    k0 = w0 * w0
    k1 = w1 * w1
    hw = H * W
    nfc0, tail0 = divmod(M * k0, _CH)
    nfc1, tail1 = divmod(M * k1, _CH)
    per0 = -(-nfc0 // _NW)
    per1 = -(-nfc1 // _NW)
    mp8 = -(-M // 8) * 8

    assert ow & (ow - 1) == 0
    ow_shift = ow.bit_length() - 1
    oh = ow
    assert (oh - 1) * STRIDE + (w0 - 1) - p0 < H
    assert (oh - 1) * STRIDE + (w1 - 1) - p1 < H

    mesh = plsc.VectorSubcoreMesh(core_axis_name="c", subcore_axis_name="s")

    @functools.partial(
        pl.kernel,
        mesh=mesh,
        compiler_params=pltpu.CompilerParams(needs_layout_passes=False),
        out_type=(
            jax.ShapeDtypeStruct((M * k0, C), jnp.float32),
            jax.ShapeDtypeStruct((M * k1, C), jnp.float32),
        ),
        scratch_types=[
            pltpu.VMEM((mp8,), jnp.int32),
            pltpu.VMEM((mp8,), jnp.int32),
            pltpu.VMEM((mp8,), jnp.int32),
            pltpu.VMEM((_CH,), jnp.int32),
            pltpu.VMEM((_CH,), jnp.int32),
            pltpu.VMEM((_CH,), jnp.int32),
            pltpu.VMEM((_CH,), jnp.int32),
            pltpu.VMEM((_CH, C), jnp.float32),
            pltpu.VMEM((_CH, C), jnp.float32),
            pltpu.VMEM((_CH, C), jnp.float32),
            pltpu.VMEM((_CH, C), jnp.float32),
            pltpu.SemaphoreType.DMA,
            pltpu.SemaphoreType.DMA,
            pltpu.SemaphoreType.DMA,
            pltpu.SemaphoreType.DMA,
        ],
    )
    def sc_fn(tab0, tab1, b_hbm, i_hbm, j_hbm, out0, out1,
              b_v, i_v, j_v, ix0, ix1, ix2, ix3,
              rows0, rows1, rows2, rows3, sem0, sem1, sem2, sem3):
        idxb = (ix0, ix1, ix2, ix3)
        rows = (rows0, rows1, rows2, rows3)
        sems = (sem0, sem1, sem2, sem3)
        wid = lax.axis_index("s") * 2 + lax.axis_index("c")
        lanes = lax.iota(jnp.int32, _LANES)
        zrv = jnp.full((_LANES,), zr, jnp.int32)
        mmax = jnp.full((_LANES,), M - 1, jnp.int32)

        pltpu.sync_copy(b_hbm, b_v)
        pltpu.sync_copy(i_hbm, i_v)
        pltpu.sync_copy(j_hbm, j_v)

        def fill_chunk(buf, c, kx, w, p, pos_v):
            kmax = jnp.full((_LANES,), kx - 1, jnp.int32)

            def grp(g, carry):
                r = c * _CH + g * _LANES + lanes
                m = jnp.minimum(r // kx, mmax)
                kk = jnp.minimum(r - m * kx, kmax)
                bv = plsc.load_gather(b_v, [m])
                pv = plsc.load_gather(pos_v, [m])
                oy = pv >> ow_shift
                ox = pv & (ow - 1)
                dy = kk // w
                dx = kk - dy * w
                y0 = oy * STRIDE + (dy - p)
                x0 = ox * STRIDE + (dx - p)
                valid = (y0 >= 0) & (x0 >= 0)
                row = jnp.where(valid, bv * hw + y0 * W + x0, zrv)
                buf[pl.ds(g * _LANES, _LANES)] = row
                return carry

            lax.fori_loop(0, _CH // _LANES, grp, 0)

        @pl.when(wid == 0)
        def _():
            for (tab, out, pos_v, kx, w, p, nfc, tail) in (
                    (tab0, out0, i_v, k0, w0, p0, nfc0, tail0),
                    (tab1, out1, j_v, k1, w1, p1, nfc1, tail1)):
                fill_chunk(ix0, nfc, kx, w, p, pos_v)
                pltpu.async_copy(tab.at[ix0], rows[0], sems[0]).wait()
                pltpu.sync_copy(rows[0].at[pl.ds(0, tail)],
                                out.at[pl.ds(nfc * _CH, tail)])

        def gather_all(tab, out, pos_v, kx, w, p, nfc, per, pending):
            start = (wid * (nfc - per)) // (_NW - 1)

            def wait64(b):
                pltpu.make_async_copy(out.at[pl.ds(0, _CH)],
                                      rows[b], sems[b]).wait()

            def fire_gather(c, b):
                pltpu.async_copy(tab.at[idxb[b]], rows[b], sems[b])

            def fire_write(c, b):
                off = pl.multiple_of((start + c) * _CH, _CH)
                pltpu.async_copy(rows[b], out.at[pl.ds(off, _CH)], sems[b])

            for b in range(4):
                if pending[b]:
                    wait64(b)
                fill_chunk(idxb[b], start + b, kx, w, p, pos_v)
                fire_gather(b, b)

            nq, _rem = divmod(per, 4)

            def quad(q, carry):
                for b in range(4):
                    c = q * 4 + b
                    wait64(b)
                    fire_write(c, b)

                    @pl.when(c + 4 < per)
                    def _():
                        fill_chunk(idxb[b], start + c + 4, kx, w, p, pos_v)
                        wait64(b)
                        fire_gather(c + 4, b)
                return carry

            lax.fori_loop(0, nq, quad, 0)
            for c in range(4 * nq, per):
                b = c % 4
                wait64(b)
                fire_write(c, b)
            return (True, True, True, True)

        pend = gather_all(tab0, out0, i_v, k0, w0, p0, nfc0, per0,
                          (False,) * 4)
        pend = gather_all(tab1, out1, j_v, k1, w1, p1, nfc1, per1, pend)
        for b in range(4):
            if pend[b]:
                pltpu.make_async_copy(out1.at[pl.ds(0, _CH)],
                                      rows[b], sems[b]).wait()

    return mp8, sc_fn


def kernel(x0, x1, b_idxes, i_idxes, j_idxes):
    w0 = W_SIZE
    e = RIGHT_EXTRA
    w1 = w0 + 2 * e
    p0 = PADDING
    p1 = PADDING + e
    N, C, H, W = x0.shape
    ow = (W + 2 * p0 - w0) // STRIDE + 1
    M = b_idxes.shape[0]

    t0 = _transpose_to_rows(x0)
    t1 = _transpose_to_rows(x1)
    zr = N * H * W

    mp8, sc_fn = _build_sc_gather(M, C, H, W, ow, w0, w1, p0, p1, zr)
    pad = mp8 - M
    b = jnp.pad(b_idxes.astype(jnp.int32), (0, pad))
    ii = jnp.pad(i_idxes.astype(jnp.int32), (0, pad))
    jj = jnp.pad(j_idxes.astype(jnp.int32), (0, pad))

    out0f, out1f = sc_fn(t0, t1, b, ii, jj)
    return (out0f.reshape(M, w0 * w0, C),
            out1f.reshape(M, w1 * w1, C))


# speedup vs baseline: 1.5121x; 1.0557x over previous
import functools

import jax
import jax.numpy as jnp
from jax import lax
from jax.experimental import pallas as pl
from jax.experimental.pallas import tpu as pltpu
from jax.experimental.pallas import tpu_sc as plsc

W_SIZE = 5
STRIDE = 4
PADDING = 2
RIGHT_EXTRA = 1

_NW = 32
_CH = 128
_LANES = 16
_RB = 8


def _transpose_to_rows(x):
    N, C, H, W = x.shape
    br = _RB * W
    nblk = (N * H) // _RB
    hb = H // _RB

    def body(x_ref, o_ref):
        i = pl.program_id(0)

        @pl.when(i < nblk)
        def _():
            blk = x_ref[0].reshape(C, br)
            o_ref[...] = jnp.transpose(blk, (1, 0))

        @pl.when(i == nblk)
        def _():
            o_ref[...] = jnp.zeros((br, C), jnp.float32)

    in_spec = pl.BlockSpec(
        (1, C, _RB, W),
        lambda i: (jnp.where(i < nblk, i // hb, 0), 0,
                   jnp.where(i < nblk, i % hb, 0), 0))
    out_spec = pl.BlockSpec((br, C), lambda i: (i, 0))
    return pl.pallas_call(
        body,
        grid=(nblk + 1,),
        in_specs=[in_spec],
        out_specs=out_spec,
        out_shape=jax.ShapeDtypeStruct(((nblk + 1) * br, C), jnp.float32),
    )(x)


def _build_sc_gather(M, C, H, W, ow, w, p, zr):
    kx = w * w
    hw = H * W
    nfc, tail = divmod(M * kx, _CH)
    per = -(-nfc // _NW)
    mp8 = -(-M // 8) * 8

    assert ow & (ow - 1) == 0
    ow_shift = ow.bit_length() - 1
    oh = ow
    assert (oh - 1) * STRIDE + (w - 1) - p < H

    mesh = plsc.VectorSubcoreMesh(core_axis_name="c", subcore_axis_name="s")

    @functools.partial(
        pl.kernel,
        mesh=mesh,
        compiler_params=pltpu.CompilerParams(needs_layout_passes=False),
        out_type=jax.ShapeDtypeStruct((M * kx, C), jnp.float32),
        scratch_types=[
            pltpu.VMEM((mp8,), jnp.int32),
            pltpu.VMEM((mp8,), jnp.int32),
            pltpu.VMEM((_CH,), jnp.int32),
            pltpu.VMEM((_CH,), jnp.int32),
            pltpu.VMEM((_CH,), jnp.int32),
            pltpu.VMEM((_CH,), jnp.int32),
            pltpu.VMEM((_CH, C), jnp.float32),
            pltpu.VMEM((_CH, C), jnp.float32),
            pltpu.VMEM((_CH, C), jnp.float32),
            pltpu.VMEM((_CH, C), jnp.float32),
            pltpu.SemaphoreType.DMA,
            pltpu.SemaphoreType.DMA,
            pltpu.SemaphoreType.DMA,
            pltpu.SemaphoreType.DMA,
        ],
    )
    def sc_fn(tab, b_hbm, p_hbm, out,
              b_v, pos_v, ix0, ix1, ix2, ix3,
              rows0, rows1, rows2, rows3, sem0, sem1, sem2, sem3):
        idxb = (ix0, ix1, ix2, ix3)
        rows = (rows0, rows1, rows2, rows3)
        sems = (sem0, sem1, sem2, sem3)
        wid = lax.axis_index("s") * 2 + lax.axis_index("c")
        lanes = lax.iota(jnp.int32, _LANES)
        zrv = jnp.full((_LANES,), zr, jnp.int32)
        mmax = jnp.full((_LANES,), M - 1, jnp.int32)

        pltpu.sync_copy(b_hbm, b_v)
        pltpu.sync_copy(p_hbm, pos_v)

        def fill_chunk(buf, c):
            kmax = jnp.full((_LANES,), kx - 1, jnp.int32)

            def grp(g, carry):
                r = c * _CH + g * _LANES + lanes
                m = jnp.minimum(r // kx, mmax)
                kk = jnp.minimum(r - m * kx, kmax)
                bv = plsc.load_gather(b_v, [m])
                pv = plsc.load_gather(pos_v, [m])
                oy = pv >> ow_shift
                ox = pv & (ow - 1)
                dy = kk // w
                dx = kk - dy * w
                y0 = oy * STRIDE + (dy - p)
                x0 = ox * STRIDE + (dx - p)
                valid = (y0 >= 0) & (x0 >= 0)
                row = jnp.where(valid, bv * hw + y0 * W + x0, zrv)
                buf[pl.ds(g * _LANES, _LANES)] = row
                return carry

            lax.fori_loop(0, _CH // _LANES, grp, 0)

        @pl.when(wid == 0)
        def _():
            fill_chunk(ix0, nfc)
            pltpu.async_copy(tab.at[ix0], rows[0], sems[0]).wait()
            pltpu.sync_copy(rows[0].at[pl.ds(0, tail)],
                            out.at[pl.ds(nfc * _CH, tail)])

        start = (wid * (nfc - per)) // (_NW - 1)

        def wait64(b):
            pltpu.make_async_copy(out.at[pl.ds(0, _CH)],
                                  rows[b], sems[b]).wait()

        def fire_gather(b):
            pltpu.async_copy(tab.at[idxb[b]], rows[b], sems[b])

        def fire_write(c, b):
            off = pl.multiple_of((start + c) * _CH, _CH)
            pltpu.async_copy(rows[b], out.at[pl.ds(off, _CH)], sems[b])

        for b in range(4):
            fill_chunk(idxb[b], start + b)
            fire_gather(b)

        nq, _rem = divmod(per, 4)

        def quad(q, carry):
            for b in range(4):
                c = q * 4 + b
                wait64(b)
                fire_write(c, b)

                @pl.when(c + 4 < per)
                def _():
                    fill_chunk(idxb[b], start + c + 4)
                    wait64(b)
                    fire_gather(b)
            return carry

        lax.fori_loop(0, nq, quad, 0)
        for c in range(4 * nq, per):
            b = c % 4
            wait64(b)
            fire_write(c, b)
        for b in range(4):
            wait64(b)

    return mp8, sc_fn


def kernel(x0, x1, b_idxes, i_idxes, j_idxes):
    w0 = W_SIZE
    e = RIGHT_EXTRA
    w1 = w0 + 2 * e
    p0 = PADDING
    p1 = PADDING + e
    N, C, H, W = x0.shape
    ow = (W + 2 * p0 - w0) // STRIDE + 1
    M = b_idxes.shape[0]

    zr = N * H * W
    mp8, sc_fn0 = _build_sc_gather(M, C, H, W, ow, w0, p0, zr)
    _, sc_fn1 = _build_sc_gather(M, C, H, W, ow, w1, p1, zr)
    pad = mp8 - M
    b = jnp.pad(b_idxes.astype(jnp.int32), (0, pad))
    ii = jnp.pad(i_idxes.astype(jnp.int32), (0, pad))
    jj = jnp.pad(j_idxes.astype(jnp.int32), (0, pad))

    t0 = _transpose_to_rows(x0)
    out0f = sc_fn0(t0, b, ii)
    t1 = _transpose_to_rows(x1)
    out1f = sc_fn1(t1, b, jj)
    return (out0f.reshape(M, w0 * w0, C),
            out1f.reshape(M, w1 * w1, C))
